# Initial kernel scaffold; baseline (speedup 1.0000x reference)
import jax
import jax.numpy as jnp
from jax.experimental import pallas as pl


def kernel(node_types, edge_index, graph_ids, emb, W_left, W_right, W_top, b_conv, gate_W, gate_b, cls_W, cls_b):
    raise NotImplementedError("write your pallas kernel here")


# speedup vs baseline: 3.3426x; 3.3426x over previous
import functools

import jax
import jax.numpy as jnp
from jax import lax
from jax.experimental import pallas as pl
from jax.experimental.pallas import tpu as pltpu
from jax.experimental.pallas import tpu_sc as plsc

N = 100000
X = 128
H = 128
G = 256
NCLS = 104

NC = 2
NS = 16
NW = NC * NS

N_PAD = 102400
E = N - 1
C_EDGE = 3328
NWIN = C_EDGE // 256
K_WIN = 256
E_PAD = NW * C_EDGE
LA = 16 + E_PAD + 16
OUTB = 128
THRESH = 120
NREC = 2 * NW

_mesh = plsc.VectorSubcoreMesh(
    core_axis_name="c", subcore_axis_name="s", num_cores=NC, num_subcores=NS)
_sc_params = pltpu.CompilerParams(needs_layout_passes=False)


def _wid():
  return lax.axis_index("s") * NC + lax.axis_index("c")


@functools.partial(
    pl.kernel,
    out_type=jax.ShapeDtypeStruct((N_PAD, X), jnp.float32),
    mesh=_mesh,
    compiler_params=_sc_params,
    scratch_types=[
        pltpu.VMEM((320,), jnp.int32),
        pltpu.VMEM((320, X), jnp.float32),
        pltpu.SemaphoreType.DMA,
    ],
)
def _emb_gather(nt_hbm, emb_hbm, h0_hbm, idx_v, rows_v, sem):
  w = _wid()
  base0 = w * (N_PAD // NW)

  def step(i, _):
    base = pl.multiple_of(base0 + i * 320, 64)
    pltpu.sync_copy(nt_hbm.at[pl.ds(base, 320)], idx_v)
    pltpu.async_copy(emb_hbm.at[idx_v], rows_v, sem).wait()
    pltpu.sync_copy(rows_v, h0_hbm.at[pl.ds(base, 320)])
    return 0

  lax.fori_loop(0, N_PAD // NW // 320, step, 0)


@functools.partial(
    pl.kernel,
    out_type=(
        jax.ShapeDtypeStruct((N_PAD, X), jnp.float32),
        jax.ShapeDtypeStruct((N_PAD, X), jnp.float32),
        jax.ShapeDtypeStruct((NW * 512,), jnp.float32),
        jax.ShapeDtypeStruct((NW * 16,), jnp.float32),
    ),
    mesh=_mesh,
    compiler_params=_sc_params,
    scratch_types=[
        pltpu.VMEM((288,), jnp.int32),
        pltpu.VMEM((K_WIN,), jnp.int32),
        pltpu.VMEM((K_WIN, X), jnp.float32),
        pltpu.VMEM((OUTB, X), jnp.float32),
        pltpu.VMEM((OUTB, X), jnp.float32),
        pltpu.VMEM((OUTB,), jnp.int32),
        pltpu.VMEM((512,), jnp.float32),
        pltpu.VMEM((16,), jnp.float32),
        pltpu.SemaphoreType.DMA,
    ],
)
def _edge_pass(src_hbm, dst_hbm, h0_hbm, sr_hbm, sl_hbm, part_hbm, meta_hbm,
               dstw, srcw, rows, out_sr, out_sl, out_dst, pstage, mstage, sem):
  w = _wid()
  t0 = w * C_EDGE
  dump = N + 16 + w
  lanes = lax.iota(jnp.int32, 16)
  zero16 = jnp.zeros((16,), jnp.float32)

  def reset_outdst():
    dv = jnp.full((16,), dump, jnp.int32)
    for j in range(OUTB // 16):
      out_dst[pl.ds(16 * j, 16)] = dv

  def drain():
    pltpu.async_copy(out_sr, sr_hbm.at[out_dst], sem).wait()
    pltpu.async_copy(out_sl, sl_hbm.at[out_dst], sem).wait()
    reset_outdst()

  def finalize(a0, a1, cntf):
    cv = jnp.full((16,), cntf, jnp.float32)
    single = cv == 1.0
    inv = 1.0 / jnp.maximum(cv - 1.0, 1.0)
    sr = [jnp.where(single, 0.5 * a0[j], a1[j] * inv) for j in range(8)]
    sl = [a0[j] - sr[j] for j in range(8)]
    return sr, sl

  reset_outdst()
  mstage[...] = jnp.where((lanes == 0) | (lanes == 8), -1.0, 0.0)

  def window(i, carry):
    base = pl.multiple_of(t0 + i * K_WIN, 8)
    pltpu.sync_copy(dst_hbm.at[pl.ds(base, 288)], dstw)
    pltpu.sync_copy(src_hbm.at[pl.ds(base + 16, K_WIN)], srcw)
    pltpu.async_copy(h0_hbm.at[srcw], rows, sem).wait()

    def edge(j, c):
      (cur, pos, isf, nout) = c[:4]
      a0 = list(c[4:12])
      a1 = list(c[12:20])
      d = dstw[pl.ds(j + 16, 16)][0]
      changed = d != cur
      live = pos > 0.0
      flush_i = changed & live & (isf == 0)
      flush_f = changed & live & (isf == 1)

      @pl.when(flush_i)
      def _():
        sr, sl = finalize(a0, a1, pos)
        ridx = jnp.full((16,), nout, jnp.int32)
        for j8 in range(8):
          cols = 16 * j8 + lanes
          plsc.store_scatter(out_sr, [ridx, cols], sr[j8])
          plsc.store_scatter(out_sl, [ridx, cols], sl[j8])
        plsc.store_scatter(out_dst, [ridx],
                           jnp.full((16,), cur, jnp.int32), mask=lanes == 0)

      @pl.when(flush_f)
      def _():
        for j8 in range(8):
          pstage[pl.ds(16 * j8, 16)] = a0[j8]
          pstage[pl.ds(128 + 16 * j8, 16)] = a1[j8]
        mv = mstage[...]
        mv = jnp.where(lanes == 0, cur.astype(jnp.float32), mv)
        mv = jnp.where(lanes == 1, pos, mv)
        mstage[...] = mv

      nout2 = nout + flush_i.astype(jnp.int32)
      do_drain = nout2 == THRESH

      @pl.when(do_drain)
      def _():
        drain()

      nout3 = jnp.where(do_drain, 0, nout2)
      posb = jnp.where(changed, 0.0, pos)
      isf2 = jnp.where(changed, 0, isf)
      rj = jnp.full((16,), j, jnp.int32)
      na0, na1 = [], []
      for j8 in range(8):
        r = plsc.load_gather(rows, [rj, 16 * j8 + lanes])
        na0.append(jnp.where(changed, zero16, a0[j8]) + r)
        na1.append(jnp.where(changed, zero16, a1[j8]) + posb * r)
      return tuple([d, posb + 1.0, isf2, nout3] + na0 + na1)

    return lax.fori_loop(0, K_WIN, edge, carry)

  pltpu.sync_copy(dst_hbm.at[pl.ds(pl.multiple_of(t0 + 8, 8), 16)],
                  dstw.at[pl.ds(0, 16)])
  cur0 = dstw[pl.ds(0, 16)][7]

  init = tuple(
      [cur0, jnp.float32(0.0), jnp.int32(1), jnp.int32(0)]
      + [zero16] * 16)
  fin = lax.fori_loop(0, NWIN, window, init)

  (cur, pos, isf, nout) = fin[:4]
  fa0 = list(fin[4:12])
  fa1 = list(fin[12:20])
  nxt = dstw[pl.ds(272, 16)][0]
  ends_after = nxt == cur
  as_first = isf == 1
  as_last = (isf == 0) & ends_after
  as_interior = (isf == 0) & jnp.logical_not(ends_after)

  @pl.when(as_first)
  def _():
    for j8 in range(8):
      pstage[pl.ds(16 * j8, 16)] = fa0[j8]
      pstage[pl.ds(128 + 16 * j8, 16)] = fa1[j8]
    mv = mstage[...]
    mv = jnp.where(lanes == 0, cur.astype(jnp.float32), mv)
    mv = jnp.where(lanes == 1, pos, mv)
    mstage[...] = mv

  @pl.when(as_last)
  def _():
    for j8 in range(8):
      pstage[pl.ds(256 + 16 * j8, 16)] = fa0[j8]
      pstage[pl.ds(384 + 16 * j8, 16)] = fa1[j8]
    mv = mstage[...]
    mv = jnp.where(lanes == 8, cur.astype(jnp.float32), mv)
    mv = jnp.where(lanes == 9, pos, mv)
    mstage[...] = mv

  @pl.when(as_interior)
  def _():
    sr, sl = finalize(fa0, fa1, pos)
    ridx = jnp.full((16,), nout, jnp.int32)
    for j8 in range(8):
      cols = 16 * j8 + lanes
      plsc.store_scatter(out_sr, [ridx, cols], sr[j8])
      plsc.store_scatter(out_sl, [ridx, cols], sl[j8])
    plsc.store_scatter(out_dst, [ridx],
                       jnp.full((16,), cur, jnp.int32), mask=lanes == 0)

  drain()
  pltpu.sync_copy(pstage, part_hbm.at[pl.ds(w * 512, 512)])
  pltpu.sync_copy(mstage, meta_hbm.at[pl.ds(w * 16, 16)])


@functools.partial(
    pl.kernel,
    out_type=(
        jax.ShapeDtypeStruct((NREC, X), jnp.float32),
        jax.ShapeDtypeStruct((NREC, X), jnp.float32),
        jax.ShapeDtypeStruct((NREC,), jnp.int32),
    ),
    mesh=_mesh,
    compiler_params=_sc_params,
    scratch_types=[
        pltpu.VMEM((NW * 512,), jnp.float32),
        pltpu.VMEM((NW * 16,), jnp.float32),
        pltpu.VMEM((NREC, X), jnp.float32),
        pltpu.VMEM((NREC, X), jnp.float32),
        pltpu.VMEM((NREC,), jnp.int32),
        pltpu.SemaphoreType.DMA,
    ],
)
def _merge(part_hbm, meta_hbm, cr_hbm, cl_hbm, bx_hbm,
           pv, mv, cr_v, cl_v, bx_v, sem):
  w = _wid()
  lanes = lax.iota(jnp.int32, 16)
  zero16 = jnp.zeros((16,), jnp.float32)

  pltpu.sync_copy(part_hbm, pv)
  pltpu.sync_copy(meta_hbm, mv)
  neg1 = jnp.full((16,), -1, jnp.int32)
  for r in range(NREC // 16):
    bx_v[pl.ds(16 * r, 16)] = neg1
  for r in range(NREC):
    for j8 in range(8):
      cr_v[r, pl.ds(16 * j8, 16)] = zero16
      cl_v[r, pl.ds(16 * j8, 16)] = zero16

  def finalize(a0, a1, cntf):
    cv = jnp.full((16,), cntf, jnp.float32)
    single = cv == 1.0
    inv = 1.0 / jnp.maximum(cv - 1.0, 1.0)
    sr = [jnp.where(single, 0.5 * a0[j], a1[j] * inv) for j in range(8)]
    sl = [a0[j] - sr[j] for j in range(8)]
    return sr, sl

  def rec(r, c):
    (have, cur, cnt, nfin) = c[:4]
    a0 = list(c[4:12])
    a1 = list(c[12:20])
    tile = r // 2
    slot = r - 2 * tile
    mb = tile * 16 + slot * 8
    pdstf = mv[pl.ds(mb, 16)][0]
    pcnt = mv[pl.ds(mb, 16)][1]
    pdst = pdstf.astype(jnp.int32)
    present = pdst >= 0
    same = present & (have == 1) & (pdst == cur)
    newopen = present & jnp.logical_not(same)
    fin_now = newopen & (have == 1)

    @pl.when(fin_now)
    def _():
      sr, sl = finalize(a0, a1, cnt)
      ridx = jnp.full((16,), nfin, jnp.int32)
      for j8 in range(8):
        cols = 16 * j8 + lanes
        plsc.store_scatter(cr_v, [ridx, cols], sr[j8])
        plsc.store_scatter(cl_v, [ridx, cols], sl[j8])
      plsc.store_scatter(bx_v, [ridx],
                         jnp.full((16,), cur, jnp.int32), mask=lanes == 0)

    nfin2 = nfin + fin_now.astype(jnp.int32)
    pb = tile * 512 + slot * 256
    na0, na1 = [], []
    for j8 in range(8):
      t0v = pv[pl.ds(pb + 16 * j8, 16)]
      t1v = pv[pl.ds(pb + 128 + 16 * j8, 16)]
      n0 = jnp.where(same, a0[j8] + t0v, jnp.where(newopen, t0v, a0[j8]))
      n1 = jnp.where(same, a1[j8] + t1v + cnt * t0v,
                     jnp.where(newopen, t1v, a1[j8]))
      na0.append(n0)
      na1.append(n1)
    cnt2 = jnp.where(same, cnt + pcnt, jnp.where(newopen, pcnt, cnt))
    cur2 = jnp.where(newopen, pdst, cur)
    have2 = jnp.where(newopen, 1, have)
    return tuple([have2, cur2, cnt2, nfin2] + na0 + na1)

  init = tuple([jnp.int32(0), jnp.int32(-1), jnp.float32(0.0), jnp.int32(0)]
               + [zero16] * 16)
  fin = lax.fori_loop(0, NREC, rec, init)
  (have, cur, cnt, nfin) = fin[:4]

  @pl.when(have == 1)
  def _():
    sr, sl = finalize(list(fin[4:12]), list(fin[12:20]), cnt)
    ridx = jnp.full((16,), nfin, jnp.int32)
    for j8 in range(8):
      cols = 16 * j8 + lanes
      plsc.store_scatter(cr_v, [ridx, cols], sr[j8])
      plsc.store_scatter(cl_v, [ridx, cols], sl[j8])
    plsc.store_scatter(bx_v, [ridx],
                       jnp.full((16,), cur, jnp.int32), mask=lanes == 0)

  @pl.when(w == 0)
  def _():
    pltpu.sync_copy(cr_v, cr_hbm)
    pltpu.sync_copy(cl_v, cl_hbm)
    pltpu.sync_copy(bx_v, bx_hbm)


BM = 512


def _conv_body(sr, sl, h0, deg, bidx, cr, cl, wr, wl, wt, b, h_out):
  i = pl.program_id(0)
  rel = bidx[...] - i * BM
  sel = (lax.broadcasted_iota(jnp.int32, (BM, NREC), 0) == rel)
  self_f = sel.astype(jnp.float32)
  hit = jnp.sum(self_f, axis=1, keepdims=True)
  sr_e = jnp.where(hit > 0.0,
                   jnp.dot(self_f, cr[...], preferred_element_type=jnp.float32),
                   sr[...])
  sl_e = jnp.where(hit > 0.0,
                   jnp.dot(self_f, cl[...], preferred_element_type=jnp.float32),
                   sl[...])
  cs = jnp.dot(sr_e, wr[...], preferred_element_type=jnp.float32)
  cs += jnp.dot(sl_e, wl[...], preferred_element_type=jnp.float32)
  cs += jnp.dot(h0[...], wt[...], preferred_element_type=jnp.float32)
  hn = jnp.maximum(cs + b[...], 0.0)
  h_out[...] = jnp.where(deg[...] > 0.0, hn, h0[...])


def _conv(sr, sl, h0, deg, bidx, cr, cl, wr, wl, wt, b):
  nb = N_PAD // BM
  blk = lambda i: (i, 0)
  cst = lambda i: (0, 0)
  return pl.pallas_call(
      _conv_body,
      grid=(nb,),
      in_specs=[
          pl.BlockSpec((BM, X), blk),
          pl.BlockSpec((BM, X), blk),
          pl.BlockSpec((BM, X), blk),
          pl.BlockSpec((BM, 1), blk),
          pl.BlockSpec((1, NREC), cst),
          pl.BlockSpec((NREC, X), cst),
          pl.BlockSpec((NREC, X), cst),
          pl.BlockSpec((X, H), cst),
          pl.BlockSpec((X, H), cst),
          pl.BlockSpec((X, H), cst),
          pl.BlockSpec((1, H), cst),
      ],
      out_specs=pl.BlockSpec((BM, H), blk),
      out_shape=jax.ShapeDtypeStruct((N_PAD, H), jnp.float32),
  )(sr, sl, h0, deg, bidx, cr, cl, wr, wl, wt, b)


NEG = -1e30


def _pool_body(h, gid, gw, gb, cw, cb, out, m_s, z_s, p_s):
  i = pl.program_id(0)
  nb = pl.num_programs(0)

  @pl.when(i == 0)
  def _():
    m_s[...] = jnp.full((G, 1), NEG, jnp.float32)
    z_s[...] = jnp.zeros((G, 1), jnp.float32)
    p_s[...] = jnp.zeros((G, H), jnp.float32)

  hb = h[...]
  s = lax.dot_general(gw[...], hb, (((1,), (1,)), ((), ()))) + gb[0, 0]
  gids = gid[0]
  mask = lax.broadcasted_iota(jnp.int32, (G, BM), 0) == gids
  bmax = jnp.max(jnp.where(mask, s, NEG), axis=1, keepdims=True)
  m_old = m_s[...]
  m_new = jnp.maximum(m_old, bmax)
  corr = jnp.where(m_new == m_old, 1.0, jnp.exp(m_old - m_new))
  m_s[...] = m_new
  e = jnp.where(mask, jnp.exp(s - m_new), 0.0)
  z_s[...] = z_s[...] * corr + jnp.sum(e, axis=1, keepdims=True)
  p_s[...] = p_s[...] * corr + jnp.dot(e, hb, preferred_element_type=jnp.float32)

  @pl.when(i == nb - 1)
  def _():
    pooled = p_s[...] / jnp.maximum(z_s[...], 1e-30)
    out[...] = lax.dot_general(
        pooled, cw[...], (((1,), (1,)), ((), ()))) + cb[...]


def _pool(h, gid, gw, gb, cw, cb):
  nb = N_PAD // BM
  blk = lambda i: (i, 0)
  cst = lambda i: (0, 0)
  return pl.pallas_call(
      _pool_body,
      grid=(nb,),
      in_specs=[
          pl.BlockSpec((BM, H), blk),
          pl.BlockSpec((1, 1, BM), lambda i: (i, 0, 0)),
          pl.BlockSpec((1, H), cst),
          pl.BlockSpec((1, 1), cst),
          pl.BlockSpec((NCLS, H), cst),
          pl.BlockSpec((1, NCLS), cst),
      ],
      out_specs=pl.BlockSpec((G, NCLS), cst),
      out_shape=jax.ShapeDtypeStruct((G, NCLS), jnp.float32),
      scratch_shapes=[
          pltpu.VMEM((G, 1), jnp.float32),
          pltpu.VMEM((G, 1), jnp.float32),
          pltpu.VMEM((G, H), jnp.float32),
      ],
  )(h, gid, gw, gb, cw, cb)


def kernel(node_types, edge_index, graph_ids, emb, W_left, W_right, W_top,
           b_conv, gate_W, gate_b, cls_W, cls_b):
  nt = node_types.astype(jnp.int32)
  src = edge_index[0].astype(jnp.int32)
  dst = edge_index[1].astype(jnp.int32)
  gid = graph_ids.astype(jnp.int32)

  nt_pad = jnp.concatenate([nt, jnp.zeros((N_PAD - N,), jnp.int32)])
  src_arr = jnp.concatenate([
      jnp.zeros((16,), jnp.int32), src, jnp.zeros((LA - 16 - E,), jnp.int32)])
  dst_arr = jnp.concatenate([
      jnp.full((16,), -1, jnp.int32), dst, jnp.full((LA - 16 - E,), N,
                                                    jnp.int32)])
  gid_pad = jnp.concatenate([gid, jnp.full((N_PAD - N,), G, jnp.int32)])

  deg = jax.ops.segment_sum(jnp.ones((E,), jnp.float32), dst, num_segments=N,
                            indices_are_sorted=True)
  deg_pad = jnp.concatenate([deg, jnp.zeros((N_PAD - N,), jnp.float32)])

  h0 = _emb_gather(nt_pad, emb)
  s_r, s_l, part, meta = _edge_pass(src_arr, dst_arr, h0)
  c_r, c_l, bidx = _merge(part, meta)
  h = _conv(s_r, s_l, h0, deg_pad.reshape(N_PAD, 1), bidx.reshape(1, NREC),
            c_r, c_l, W_right, W_left, W_top, b_conv)
  logits = _pool(h, gid_pad.reshape(N_PAD // BM, 1, BM), gate_W,
                 gate_b.reshape(1, 1), cls_W, cls_b.reshape(1, NCLS))
  return logits
